# baseline (speedup 1.0000x reference)
import functools
import jax
import jax.numpy as jnp
from jax import lax
from jax.experimental import pallas as pl
from jax.experimental.pallas import tpu as pltpu
from jax.experimental.pallas import tpu_sc as plsc

_K = 8
_L = 16


def _logits_body(x_ref, w_ref, out_ref):
    logits = jnp.dot(x_ref[...], w_ref[...], preferred_element_type=jnp.float32)
    out_ref[...] = logits.T


def _make_sc_router(BT, E):
    info = plsc.get_sparse_core_info()
    NW = info.num_cores * info.num_subcores
    ntok = BT // NW
    ngrp = ntok // _L
    mesh = plsc.VectorSubcoreMesh(core_axis_name="c", subcore_axis_name="s")

    @functools.partial(
        pl.kernel,
        mesh=mesh,
        out_type=[
            jax.ShapeDtypeStruct((BT * _K,), jnp.float32),
            jax.ShapeDtypeStruct((BT * _K,), jnp.int32),
        ],
        scratch_types=[
            pltpu.VMEM((E, ntok), jnp.float32),
            pltpu.VMEM((E * _L,), jnp.float32),
            pltpu.VMEM((ntok * _K,), jnp.float32),
            pltpu.VMEM((ntok * _K,), jnp.int32),
        ],
        compiler_params=pltpu.CompilerParams(needs_layout_passes=False),
    )
    def sc_router(lt_hbm, out_w_hbm, out_i_hbm, lbuf, pbuf, wbuf, ibuf):
        wid = lax.axis_index("s") * info.num_cores + lax.axis_index("c")
        base = wid * ntok
        pltpu.sync_copy(lt_hbm.at[:, pl.ds(base, ntok)], lbuf)
        lane = lax.iota(jnp.int32, _L)
        lane_k = lane * _K

        def group_body(g, carry):
            gbase = g * _L
            l = [lbuf[e, pl.ds(gbase, _L)] for e in range(E)]
            m = l[0]
            for e in range(1, E):
                m = jnp.maximum(m, l[e])
            ev = [jnp.exp(l[e] - m) for e in range(E)]
            z = ev[0]
            for e in range(1, E):
                z = z + ev[e]
            r = 1.0 / z
            for e in range(E):
                pbuf[pl.ds(e * _L, _L)] = ev[e] * r

            vk = []
            ik = []
            for k in range(_K):
                vals = [pbuf[pl.ds(e * _L, _L)] for e in range(E)]
                idxs = [jnp.full((_L,), e, jnp.int32) for e in range(E)]
                n = E
                while n > 1:
                    nv, ni = [], []
                    for j in range(0, n, 2):
                        ge = vals[j] >= vals[j + 1]
                        nv.append(jnp.where(ge, vals[j], vals[j + 1]))
                        ni.append(jnp.where(ge, idxs[j], idxs[j + 1]))
                    vals, idxs, n = nv, ni, n // 2
                vk.append(vals[0])
                ik.append(idxs[0])
                plsc.store_scatter(
                    pbuf,
                    [idxs[0] * _L + lane],
                    jnp.full((_L,), -1.0, jnp.float32),
                )

            e2 = [jnp.exp(vk[k] - vk[0]) for k in range(_K)]
            s2 = e2[0]
            for k in range(1, _K):
                s2 = s2 + e2[k]
            r2 = 1.0 / s2
            obase = g * (_L * _K)
            for k in range(_K):
                oidx = lane_k + (obase + k)
                plsc.store_scatter(wbuf, [oidx], e2[k] * r2)
                plsc.store_scatter(ibuf, [oidx], ik[k])
            return carry

        lax.fori_loop(0, ngrp, group_body, 0)
        pltpu.sync_copy(wbuf, out_w_hbm.at[pl.ds(base * _K, ntok * _K)])
        pltpu.sync_copy(ibuf, out_i_hbm.at[pl.ds(base * _K, ntok * _K)])

    return sc_router


def kernel(x, kernel_DE):
    B, T, D = x.shape
    E = kernel_DE.shape[1]
    BT = B * T
    bt = 2048
    x2 = x.reshape(BT, D)

    lt = pl.pallas_call(
        _logits_body,
        grid=(BT // bt,),
        in_specs=[
            pl.BlockSpec((bt, D), lambda i: (i, 0)),
            pl.BlockSpec((D, E), lambda i: (0, 0)),
        ],
        out_specs=pl.BlockSpec((E, bt), lambda i: (0, i)),
        out_shape=jax.ShapeDtypeStruct((E, BT), jnp.float32),
    )(x2, kernel_DE)

    w_flat, i_flat = _make_sc_router(BT, E)(lt)
    return w_flat.reshape(B, T, _K), i_flat.reshape(B, T, _K)


# speedup vs baseline: 1.2831x; 1.2831x over previous
import functools
import jax
import jax.numpy as jnp
from jax import lax
from jax.experimental import pallas as pl
from jax.experimental.pallas import tpu as pltpu
from jax.experimental.pallas import tpu_sc as plsc

_K = 8
_L = 16


def _logits_body(x_ref, w_ref, out_ref):
    logits = jnp.dot(x_ref[...], w_ref[...], preferred_element_type=jnp.float32)
    out_ref[...] = logits.T


def _make_sc_router(BT, E):
    info = plsc.get_sparse_core_info()
    NW = info.num_cores * info.num_subcores
    ntok = BT // NW
    ngrp = ntok // _L
    mesh = plsc.VectorSubcoreMesh(core_axis_name="c", subcore_axis_name="s")

    @functools.partial(
        pl.kernel,
        mesh=mesh,
        out_type=[
            jax.ShapeDtypeStruct((_K, BT), jnp.float32),
            jax.ShapeDtypeStruct((_K, BT), jnp.int32),
        ],
        scratch_types=[
            pltpu.VMEM((E, ntok), jnp.float32),
            pltpu.VMEM((E * _L,), jnp.float32),
            pltpu.VMEM((_K, ntok), jnp.float32),
            pltpu.VMEM((_K, ntok), jnp.int32),
        ],
        compiler_params=pltpu.CompilerParams(needs_layout_passes=False),
    )
    def sc_router(lt_hbm, out_w_hbm, out_i_hbm, lbuf, pbuf, wbuf, ibuf):
        wid = lax.axis_index("s") * info.num_cores + lax.axis_index("c")
        base = wid * ntok
        pltpu.sync_copy(lt_hbm.at[:, pl.ds(base, ntok)], lbuf)
        lane = lax.iota(jnp.int32, _L)

        def group_body(g, carry):
            gbase = g * _L
            l = [lbuf[e, pl.ds(gbase, _L)] for e in range(E)]
            m = l[0]
            for e in range(1, E):
                m = jnp.maximum(m, l[e])
            ev = [jnp.exp(l[e] - m) for e in range(E)]
            z = ev[0]
            for e in range(1, E):
                z = z + ev[e]
            r = 1.0 / z
            for e in range(E):
                pbuf[pl.ds(e * _L, _L)] = ev[e] * r

            vk = []
            ik = []
            for k in range(_K):
                vals = [pbuf[pl.ds(e * _L, _L)] for e in range(E)]
                idxs = [jnp.full((_L,), e, jnp.int32) for e in range(E)]
                n = E
                while n > 1:
                    nv, ni = [], []
                    for j in range(0, n, 2):
                        ge = vals[j] >= vals[j + 1]
                        nv.append(jnp.where(ge, vals[j], vals[j + 1]))
                        ni.append(jnp.where(ge, idxs[j], idxs[j + 1]))
                    vals, idxs, n = nv, ni, n // 2
                vk.append(vals[0])
                ik.append(idxs[0])
                plsc.store_scatter(
                    pbuf,
                    [idxs[0] * _L + lane],
                    jnp.full((_L,), -1.0, jnp.float32),
                )

            e2 = [jnp.exp(vk[k] - vk[0]) for k in range(_K)]
            s2 = e2[0]
            for k in range(1, _K):
                s2 = s2 + e2[k]
            r2 = 1.0 / s2
            for k in range(_K):
                wbuf[k, pl.ds(gbase, _L)] = e2[k] * r2
                ibuf[k, pl.ds(gbase, _L)] = ik[k]
            return carry

        lax.fori_loop(0, ngrp, group_body, 0)
        pltpu.sync_copy(wbuf, out_w_hbm.at[:, pl.ds(base, ntok)])
        pltpu.sync_copy(ibuf, out_i_hbm.at[:, pl.ds(base, ntok)])

    return sc_router


def kernel(x, kernel_DE):
    B, T, D = x.shape
    E = kernel_DE.shape[1]
    BT = B * T
    bt = 2048
    x2 = x.reshape(BT, D)

    lt = pl.pallas_call(
        _logits_body,
        grid=(BT // bt,),
        in_specs=[
            pl.BlockSpec((bt, D), lambda i: (i, 0)),
            pl.BlockSpec((D, E), lambda i: (0, 0)),
        ],
        out_specs=pl.BlockSpec((E, bt), lambda i: (0, i)),
        out_shape=jax.ShapeDtypeStruct((E, BT), jnp.float32),
    )(x2, kernel_DE)

    w_t, i_t = _make_sc_router(BT, E)(lt)
    return w_t.T.reshape(B, T, _K), i_t.T.reshape(B, T, _K)
